# baseline (speedup 1.0000x reference)
import functools

import jax
import jax.numpy as jnp
from jax import lax
from jax.experimental import pallas as pl
from jax.experimental.pallas import tpu as pltpu
from jax.experimental.pallas import tpu_sc as plsc

_ROWS = 64
_N = 4096
_LANES = 16
_NCHUNK = _N // _LANES
_UNROLL = 4
_NGROUP = _NCHUNK // _UNROLL
_NWORKERS = 32
_ROWS_PER_W = _ROWS // _NWORKERS
_SENTINEL = -1e30

_mesh = plsc.VectorSubcoreMesh(core_axis_name="c", subcore_axis_name="s")


@functools.partial(
    pl.kernel,
    mesh=_mesh,
    compiler_params=pltpu.CompilerParams(needs_layout_passes=False,
                                         use_tc_tiling_on_sc=True,
                                         disable_bounds_checks=True),
    out_type=jax.ShapeDtypeStruct((_ROWS, _N), jnp.float32),
    scratch_types=[
        pltpu.VMEM((_N,), jnp.float32),
        pltpu.VMEM((_N,), jnp.float32),
        pltpu.VMEM((_N + _LANES,), jnp.float32),
        pltpu.VMEM((_N + _LANES,), jnp.int32),
        pltpu.VMEM((_N + _LANES,), jnp.float32),
        pltpu.VMEM((_N + _LANES,), jnp.float32),
        pltpu.SemaphoreType.DMA,
        pltpu.SemaphoreType.DMA,
        pltpu.SemaphoreType.DMA,
        pltpu.SemaphoreType.DMA,
    ],
)
def _sparsemax_sc(x_hbm, out_hbm, row_v0, row_v1, comp_v, idx_v, out_v0,
                  out_v1, ld_sem0, ld_sem1, st_sem0, st_sem1):
    wid = lax.axis_index("s") * 2 + lax.axis_index("c")
    row0 = wid * _ROWS_PER_W
    row1 = row0 + 1

    ld0 = pltpu.make_async_copy(x_hbm.at[row0], row_v0, ld_sem0)
    ld1 = pltpu.make_async_copy(x_hbm.at[row1], row_v1, ld_sem1)
    ld0.start()
    ld1.start()

    lane = lax.iota(jnp.int32, _LANES)
    zerof = jnp.zeros((_LANES,), jnp.float32)
    zeroi = jnp.zeros((_LANES,), jnp.int32)
    sentv = jnp.full((_LANES,), _SENTINEL, jnp.float32)

    def zero_body(i, carry):
        b = i * (_UNROLL * _LANES)
        for j in range(_UNROLL):
            s = pl.ds(b + j * _LANES, _LANES)
            out_v0[s] = zerof
            out_v1[s] = zerof
        return carry

    lax.fori_loop(0, _NGROUP, zero_body, jnp.int32(0))

    def process_row(row_v, out_v):
        def max_body(i, accs):
            a0, a1 = accs
            b = i * (_UNROLL * _LANES)
            vs = [row_v[pl.ds(b + j * _LANES, _LANES)] for j in range(_UNROLL)]
            h = _UNROLL // 2
            for v in vs[:h]:
                a0 = jnp.maximum(a0, v)
            for v in vs[h:]:
                a1 = jnp.maximum(a1, v)
            return a0, a1

        a0, a1 = lax.fori_loop(0, _NGROUP, max_body, (sentv, sentv))
        mxv = jnp.full((_LANES,), jnp.max(jnp.maximum(a0, a1)), jnp.float32)

        def comp_body(i, off):
            b = i * (_UNROLL * _LANES)
            ys = [row_v[pl.ds(b + j * _LANES, _LANES)] - mxv
                  for j in range(_UNROLL)]
            ms = [y > -1.0 for y in ys]
            pcs = [plsc.all_reduce_population_count(m)[0] for m in ms]
            offs = []
            for j in range(_UNROLL):
                offs.append(off)
                off = off + pcs[j]
            for j in range(_UNROLL):
                plsc.store_compressed(comp_v.at[pl.ds(offs[j], _LANES)],
                                      ys[j], mask=ms[j])
                plsc.store_compressed(idx_v.at[pl.ds(offs[j], _LANES)],
                                      lane + (b + j * _LANES), mask=ms[j])
            return off

        mcount = lax.fori_loop(0, _NGROUP, comp_body, jnp.int32(0))
        comp_v[pl.ds(mcount, _LANES)] = sentv
        idx_v[pl.ds(mcount, _LANES)] = jnp.full((_LANES,), _N, jnp.int32)
        nch = lax.shift_right_logical(mcount + (_LANES - 1), 4)

        def sum_cnt(tv):
            def body(i, carry):
                ssum, cnt = carry
                v = comp_v[pl.ds(i * _LANES, _LANES)]
                act = v > tv
                return (ssum + jnp.where(act, v, 0.0),
                        cnt + jnp.where(act, 1, 0))

            return lax.fori_loop(0, nch, body, (zerof, zeroi))

        sv, cv = sum_cnt(jnp.full((_LANES,), -1.0, jnp.float32))
        k = jnp.sum(cv)
        tauv = (jnp.full((_LANES,), jnp.sum(sv) - 1.0, jnp.float32)
                / jnp.full((_LANES,), k, jnp.int32).astype(jnp.float32))

        def w_cond(carry):
            _, kprev, kcur = carry
            return kcur != kprev

        def w_body(carry):
            tv, _, kcur = carry
            sv, cv = sum_cnt(tv)
            k2 = jnp.sum(cv)
            tv2 = (jnp.full((_LANES,), jnp.sum(sv) - 1.0, jnp.float32)
                   / jnp.full((_LANES,), k2, jnp.int32).astype(jnp.float32))
            return tv2, kcur, k2

        tauv, _, _ = lax.while_loop(w_cond, w_body, (tauv, jnp.int32(-1), k))

        def out_body(i, carry):
            s = pl.ds(i * _LANES, _LANES)
            o = jnp.maximum(comp_v[s] - tauv, 0.0)
            plsc.store_scatter(out_v, [idx_v[s]], o)
            return carry

        lax.fori_loop(0, nch, out_body, jnp.int32(0))

    ld0.wait()
    process_row(row_v0, out_v0)
    st0 = pltpu.make_async_copy(out_v0.at[pl.ds(0, _N)], out_hbm.at[row0],
                                st_sem0)
    st0.start()
    ld1.wait()
    process_row(row_v1, out_v1)
    st1 = pltpu.make_async_copy(out_v1.at[pl.ds(0, _N)], out_hbm.at[row1],
                                st_sem1)
    st1.start()
    st0.wait()
    st1.wait()


def kernel(input):
    return _sparsemax_sc(input)


# speedup vs baseline: 1.0373x; 1.0373x over previous
import functools

import jax
import jax.numpy as jnp
from jax import lax
from jax.experimental import pallas as pl
from jax.experimental.pallas import tpu as pltpu
from jax.experimental.pallas import tpu_sc as plsc

_ROWS = 64
_N = 4096
_LANES = 16
_NCHUNK = _N // _LANES
_UNROLL = 8
_NGROUP = _NCHUNK // _UNROLL
_NWORKERS = 32
_ROWS_PER_W = _ROWS // _NWORKERS
_SENTINEL = -1e30

_mesh = plsc.VectorSubcoreMesh(core_axis_name="c", subcore_axis_name="s")


@functools.partial(
    pl.kernel,
    mesh=_mesh,
    compiler_params=pltpu.CompilerParams(needs_layout_passes=False,
                                         use_tc_tiling_on_sc=True,
                                         disable_bounds_checks=True),
    out_type=jax.ShapeDtypeStruct((_ROWS, _N), jnp.float32),
    scratch_types=[
        pltpu.VMEM((_N,), jnp.float32),
        pltpu.VMEM((_N,), jnp.float32),
        pltpu.VMEM((_N + _LANES,), jnp.float32),
        pltpu.VMEM((_N + _LANES,), jnp.int32),
        pltpu.VMEM((_N + _LANES,), jnp.float32),
        pltpu.VMEM((_N + _LANES,), jnp.float32),
        pltpu.SemaphoreType.DMA,
        pltpu.SemaphoreType.DMA,
        pltpu.SemaphoreType.DMA,
        pltpu.SemaphoreType.DMA,
    ],
)
def _sparsemax_sc(x_hbm, out_hbm, row_v0, row_v1, comp_v, idx_v, out_v0,
                  out_v1, ld_sem0, ld_sem1, st_sem0, st_sem1):
    wid = lax.axis_index("s") * 2 + lax.axis_index("c")
    row0 = wid * _ROWS_PER_W
    row1 = row0 + 1

    ld0 = pltpu.make_async_copy(x_hbm.at[row0], row_v0, ld_sem0)
    ld1 = pltpu.make_async_copy(x_hbm.at[row1], row_v1, ld_sem1)
    ld0.start()
    ld1.start()

    lane = lax.iota(jnp.int32, _LANES)
    zerof = jnp.zeros((_LANES,), jnp.float32)
    zeroi = jnp.zeros((_LANES,), jnp.int32)
    sentv = jnp.full((_LANES,), _SENTINEL, jnp.float32)

    def zero_body(i, carry):
        b = i * (_UNROLL * _LANES)
        for j in range(_UNROLL):
            s = pl.ds(b + j * _LANES, _LANES)
            out_v0[s] = zerof
            out_v1[s] = zerof
        return carry

    lax.fori_loop(0, _NGROUP, zero_body, jnp.int32(0))

    def process_row(row_v, out_v):
        def max_body(i, accs):
            a0, a1 = accs
            b = i * (_UNROLL * _LANES)
            vs = [row_v[pl.ds(b + j * _LANES, _LANES)] for j in range(_UNROLL)]
            h = _UNROLL // 2
            for v in vs[:h]:
                a0 = jnp.maximum(a0, v)
            for v in vs[h:]:
                a1 = jnp.maximum(a1, v)
            return a0, a1

        a0, a1 = lax.fori_loop(0, _NGROUP, max_body, (sentv, sentv))
        mxv = jnp.full((_LANES,), jnp.max(jnp.maximum(a0, a1)), jnp.float32)

        def comp_body(i, off):
            b = i * (_UNROLL * _LANES)
            ys = [row_v[pl.ds(b + j * _LANES, _LANES)] - mxv
                  for j in range(_UNROLL)]
            ms = [y > -1.0 for y in ys]
            pcs = [plsc.all_reduce_population_count(m)[0] for m in ms]
            offs = []
            for j in range(_UNROLL):
                offs.append(off)
                off = off + pcs[j]
            for j in range(_UNROLL):
                plsc.store_compressed(comp_v.at[pl.ds(offs[j], _LANES)],
                                      ys[j], mask=ms[j])
                plsc.store_compressed(idx_v.at[pl.ds(offs[j], _LANES)],
                                      lane + (b + j * _LANES), mask=ms[j])
            return off

        mcount = lax.fori_loop(0, _NGROUP, comp_body, jnp.int32(0))
        comp_v[pl.ds(mcount, _LANES)] = sentv
        idx_v[pl.ds(mcount, _LANES)] = jnp.full((_LANES,), _N, jnp.int32)
        nch = lax.shift_right_logical(mcount + (_LANES - 1), 4)

        def sum_cnt(tv):
            def body(i, carry):
                ssum, cnt = carry
                v = comp_v[pl.ds(i * _LANES, _LANES)]
                act = v > tv
                return (ssum + jnp.where(act, v, 0.0),
                        cnt + jnp.where(act, 1, 0))

            return lax.fori_loop(0, nch, body, (zerof, zeroi))

        sv, cv = sum_cnt(jnp.full((_LANES,), -1.0, jnp.float32))
        k = jnp.sum(cv)
        tauv = (jnp.full((_LANES,), jnp.sum(sv) - 1.0, jnp.float32)
                / jnp.full((_LANES,), k, jnp.int32).astype(jnp.float32))

        def w_cond(carry):
            _, kprev, kcur = carry
            return kcur != kprev

        def w_body(carry):
            tv, _, kcur = carry
            sv, cv = sum_cnt(tv)
            k2 = jnp.sum(cv)
            tv2 = (jnp.full((_LANES,), jnp.sum(sv) - 1.0, jnp.float32)
                   / jnp.full((_LANES,), k2, jnp.int32).astype(jnp.float32))
            return tv2, kcur, k2

        tauv, _, _ = lax.while_loop(w_cond, w_body, (tauv, jnp.int32(-1), k))

        def out_body(i, carry):
            s = pl.ds(i * _LANES, _LANES)
            o = jnp.maximum(comp_v[s] - tauv, 0.0)
            plsc.store_scatter(out_v, [idx_v[s]], o)
            return carry

        lax.fori_loop(0, nch, out_body, jnp.int32(0))

    ld0.wait()
    process_row(row_v0, out_v0)
    st0 = pltpu.make_async_copy(out_v0.at[pl.ds(0, _N)], out_hbm.at[row0],
                                st_sem0)
    st0.start()
    ld1.wait()
    process_row(row_v1, out_v1)
    st1 = pltpu.make_async_copy(out_v1.at[pl.ds(0, _N)], out_hbm.at[row1],
                                st_sem1)
    st1.start()
    st0.wait()
    st1.wait()


def kernel(input):
    return _sparsemax_sc(input)
